# baseline (speedup 1.0000x reference)
import functools

import jax
import jax.numpy as jnp
from jax import lax
from jax.experimental import pallas as pl
from jax.experimental.pallas import tpu as pltpu
from jax.experimental.pallas import tpu_sc as plsc

NC, NS, LANES = 2, 16, 16
NW = NC * NS

WORD_DIM = 64
POS_DIM = 64
OUT_DIM = WORD_DIM + POS_DIM

IDX_MINOR = 128
K = 1
CHUNK = K * IDX_MINOR
NBUF = 4


TBLK = 16384


def _transpose_pad_kernel(wt_ref, out_ref):
    t = wt_ref[...].T
    out_ref[...] = jnp.concatenate([t, jnp.zeros_like(t)], axis=-1)


def _emb_kernel(bt, word_hbm, pos_hbm, wtab_hbm, ptab_hbm, out_hbm,
                widx_v, pidx_v, wrows_v, prows_v,
                sem_idx, sem_rows, sem_wr):
    per_tile = bt // NW
    n_chunks = per_tile // CHUNK
    wid = lax.axis_index("s") * NC + lax.axis_index("c")
    tile_row0 = wid * (per_tile // IDX_MINOR)


    def idx_copies(g, b):
        row0 = tile_row0 + g * K
        return (
            pltpu.make_async_copy(word_hbm.at[pl.ds(row0, K)], widx_v.at[b],
                                  sem_idx.at[b]),
            pltpu.make_async_copy(pos_hbm.at[pl.ds(row0, K)], pidx_v.at[b],
                                  sem_idx.at[b]),
        )

    def write_copies(g, b):
        base = (tile_row0 + g * K) * IDX_MINOR
        return (
            pltpu.make_async_copy(
                wrows_v.at[b], out_hbm.at[pl.ds(base, CHUNK), pl.ds(0, WORD_DIM)],
                sem_wr.at[b]),
            pltpu.make_async_copy(
                prows_v.at[b],
                out_hbm.at[pl.ds(base, CHUNK), pl.ds(WORD_DIM, POS_DIM)],
                sem_wr.at[b]),
        )

    def fire_gathers(b):
        copies = []
        for j in range(K):
            copies.append(pltpu.make_async_copy(
                wtab_hbm.at[widx_v.at[b, j]],
                wrows_v.at[b, pl.ds(j * IDX_MINOR, IDX_MINOR)], sem_rows))
            copies.append(pltpu.make_async_copy(
                ptab_hbm.at[pidx_v.at[b, j]],
                prows_v.at[b, pl.ds(j * IDX_MINOR, IDX_MINOR)], sem_rows))
        for cp in copies:
            cp.start()
        return copies

    for b in range(NBUF):
        for cp in idx_copies(b, b):
            cp.start()

    def body(g2, carry):
        for b in range(NBUF):
            g = NBUF * g2 + b
            for cp in idx_copies(g, b):
                cp.wait()

            @pl.when(g2 > 0)
            def _():
                for cp in write_copies(g, b):
                    cp.wait()

            gathers = fire_gathers(b)
            for cp in gathers:
                cp.wait()

            @pl.when(g2 < n_chunks // NBUF - 1)
            def _():
                for cp in idx_copies(g + NBUF, b):
                    cp.start()

            for cp in write_copies(g, b):
                cp.start()
        return carry

    lax.fori_loop(0, n_chunks // NBUF, body, 0)

    for b in range(NBUF):
        g = n_chunks - NBUF + b
        for cp in write_copies(g, b):
            cp.wait()


def kernel(word, pos, word_table, pos_table):
    b, l = word.shape
    bt = b * l
    per_tile = bt // NW
    vocab = word_table.shape[0]
    wt_pad = pl.pallas_call(
        _transpose_pad_kernel,
        grid=(pl.cdiv(vocab, TBLK),),
        in_specs=[pl.BlockSpec((WORD_DIM, TBLK), lambda i: (0, i))],
        out_specs=pl.BlockSpec((TBLK, IDX_MINOR), lambda i: (i, 0)),
        out_shape=jax.ShapeDtypeStruct((vocab, IDX_MINOR), jnp.float32),
    )(word_table.T)
    wt_view = wt_pad.reshape(2 * vocab, WORD_DIM)
    word_flat = (word.astype(jnp.int32) * 2).reshape(bt // IDX_MINOR, IDX_MINOR)
    pos_size = pos_table.shape[0]
    ptab_rep = jnp.broadcast_to(
        pos_table[None], (NW,) + pos_table.shape).reshape(NW * pos_size,
                                                          pos_table.shape[1])
    rows_per_worker = per_tile // l
    rep_off = (jnp.arange(b, dtype=jnp.int32) // rows_per_worker) * pos_size
    pos_flat = (pos.astype(jnp.int32) + rep_off[:, None]).reshape(
        bt // IDX_MINOR, IDX_MINOR)

    mesh = plsc.VectorSubcoreMesh(core_axis_name="c", subcore_axis_name="s")
    out = pl.kernel(
        functools.partial(_emb_kernel, bt),
        out_type=jax.ShapeDtypeStruct((bt, OUT_DIM), jnp.float32),
        mesh=mesh,
        compiler_params=pltpu.CompilerParams(use_tc_tiling_on_sc=False),
        scratch_types=[
            pltpu.VMEM((NBUF, K, IDX_MINOR), jnp.int32),
            pltpu.VMEM((NBUF, K, IDX_MINOR), jnp.int32),
            pltpu.VMEM((NBUF, CHUNK, WORD_DIM), jnp.float32),
            pltpu.VMEM((NBUF, CHUNK, POS_DIM), jnp.float32),
            pltpu.SemaphoreType.DMA((NBUF,)),
            pltpu.SemaphoreType.DMA,
            pltpu.SemaphoreType.DMA((NBUF,)),
        ],
    )(word_flat, pos_flat, wt_view, ptab_rep)
    return out.reshape(b, l, OUT_DIM)


# speedup vs baseline: 1.1159x; 1.1159x over previous
import functools

import jax
import jax.numpy as jnp
from jax import lax
from jax.experimental import pallas as pl
from jax.experimental.pallas import tpu as pltpu
from jax.experimental.pallas import tpu_sc as plsc

NC, NS, LANES = 2, 16, 16
NW = NC * NS

WORD_DIM = 64
POS_DIM = 64
OUT_DIM = WORD_DIM + POS_DIM

IDX_MINOR = 128
K = 4
CHUNK = K * IDX_MINOR
NBUF = 2

TBLK = 16384


def _transpose_pad_kernel(wt_ref, out_ref):
    t = wt_ref[...].T
    out_ref[...] = jnp.concatenate([t, jnp.zeros_like(t)], axis=-1)


def _half_kernel(bt, col_off, idx_hbm, tab_hbm, out_hbm,
                 idx_v, rows_v, sem_idx, sem_rows, sem_wr):
    per_tile = bt // NW
    n_chunks = per_tile // CHUNK
    wid = lax.axis_index("s") * NC + lax.axis_index("c")
    tile_row0 = wid * (per_tile // IDX_MINOR)

    def idx_copy(g, b):
        row0 = tile_row0 + g * K
        return pltpu.make_async_copy(
            idx_hbm.at[pl.ds(row0, K)], idx_v.at[b], sem_idx.at[b])

    def write_copy(g, b):
        base = (tile_row0 + g * K) * IDX_MINOR
        return pltpu.make_async_copy(
            rows_v.at[b],
            out_hbm.at[pl.ds(base, CHUNK), pl.ds(col_off, WORD_DIM)],
            sem_wr.at[b])

    def fire_gathers(b):
        copies = [
            pltpu.make_async_copy(
                tab_hbm.at[idx_v.at[b, j]],
                rows_v.at[b, pl.ds(j * IDX_MINOR, IDX_MINOR)], sem_rows)
            for j in range(K)
        ]
        for cp in copies:
            cp.start()
        return copies

    for b in range(NBUF):
        idx_copy(b, b).start()

    def body(g2, carry):
        for b in range(NBUF):
            g = NBUF * g2 + b
            idx_copy(g, b).wait()

            @pl.when(g2 > 0)
            def _():
                write_copy(g, b).wait()

            gathers = fire_gathers(b)
            for cp in gathers:
                cp.wait()

            @pl.when(g2 < n_chunks // NBUF - 1)
            def _():
                idx_copy(g + NBUF, b).start()

            write_copy(g, b).start()
        return carry

    lax.fori_loop(0, n_chunks // NBUF, body, 0)

    for b in range(NBUF):
        write_copy(n_chunks - NBUF + b, b).wait()


def kernel(word, pos, word_table, pos_table):
    b, l = word.shape
    bt = b * l
    per_tile = bt // NW
    vocab = word_table.shape[0]
    wt_pad = pl.pallas_call(
        _transpose_pad_kernel,
        grid=(pl.cdiv(vocab, TBLK),),
        in_specs=[pl.BlockSpec((WORD_DIM, TBLK), lambda i: (0, i))],
        out_specs=pl.BlockSpec((TBLK, IDX_MINOR), lambda i: (i, 0)),
        out_shape=jax.ShapeDtypeStruct((vocab, IDX_MINOR), jnp.float32),
    )(word_table.T)
    wt_view = wt_pad.reshape(2 * vocab, WORD_DIM)
    word_flat = (word.astype(jnp.int32) * 2).reshape(bt // IDX_MINOR, IDX_MINOR)

    pos_size = pos_table.shape[0]
    ptab_rep = jnp.broadcast_to(
        pos_table[None], (NW,) + pos_table.shape).reshape(NW * pos_size,
                                                          pos_table.shape[1])
    rows_per_worker = per_tile // l
    rep_off = (jnp.arange(b, dtype=jnp.int32) // rows_per_worker) * pos_size
    pos_flat = (pos.astype(jnp.int32) + rep_off[:, None]).reshape(
        bt // IDX_MINOR, IDX_MINOR)

    mesh = plsc.VectorSubcoreMesh(core_axis_name="c", subcore_axis_name="s")
    scratch = [
        pltpu.VMEM((NBUF, K, IDX_MINOR), jnp.int32),
        pltpu.VMEM((NBUF, CHUNK, WORD_DIM), jnp.float32),
        pltpu.SemaphoreType.DMA((NBUF,)),
        pltpu.SemaphoreType.DMA,
        pltpu.SemaphoreType.DMA((NBUF,)),
    ]
    pos_out = pl.kernel(
        functools.partial(_half_kernel, bt, WORD_DIM),
        out_type=jax.ShapeDtypeStruct((bt, OUT_DIM), jnp.float32),
        mesh=mesh,
        compiler_params=pltpu.CompilerParams(use_tc_tiling_on_sc=False),
        scratch_types=scratch,
        name="pos_half",
    )(pos_flat, ptab_rep)
    out_ref = jax.new_ref(pos_out)
    pl.kernel(
        functools.partial(_half_kernel, bt, 0),
        out_type=(),
        mesh=mesh,
        compiler_params=pltpu.CompilerParams(use_tc_tiling_on_sc=False),
        scratch_types=scratch,
        name="word_half",
    )(word_flat, wt_view, out_ref)
    return out_ref[...].reshape(b, l, OUT_DIM)


# speedup vs baseline: 1.2413x; 1.1124x over previous
import functools

import jax
import jax.numpy as jnp
from jax import lax
from jax.experimental import pallas as pl
from jax.experimental.pallas import tpu as pltpu
from jax.experimental.pallas import tpu_sc as plsc

NC, NS, LANES = 2, 16, 16
NW = NC * NS

WORD_DIM = 64
POS_DIM = 64
OUT_DIM = WORD_DIM + POS_DIM

IDX_MINOR = 128
K = 4
CHUNK = K * IDX_MINOR
NBUF = 2

TBLK = 16384
TH = TBLK // 2


def _transpose_pad_kernel(wt_ref, out_ref):
    t = wt_ref[...].T
    out_ref[...] = jnp.concatenate([t[:TH], t[TH:]], axis=-1)


def _half_kernel(bt, col_off, idx_hbm, tab_hbm, out_hbm,
                 idx_v, rows_v, sem_idx, sem_rows, sem_wr):
    per_tile = bt // NW
    n_chunks = per_tile // CHUNK
    wid = lax.axis_index("s") * NC + lax.axis_index("c")
    tile_row0 = wid * (per_tile // IDX_MINOR)

    def idx_copy(g, b):
        row0 = tile_row0 + g * K
        return pltpu.make_async_copy(
            idx_hbm.at[pl.ds(row0, K)], idx_v.at[b], sem_idx.at[b])

    def write_copy(g, b):
        base = (tile_row0 + g * K) * IDX_MINOR
        return pltpu.make_async_copy(
            rows_v.at[b],
            out_hbm.at[pl.ds(base, CHUNK), pl.ds(col_off, WORD_DIM)],
            sem_wr.at[b])

    def fire_gathers(b):
        copies = [
            pltpu.make_async_copy(
                tab_hbm.at[idx_v.at[b, j]],
                rows_v.at[b, pl.ds(j * IDX_MINOR, IDX_MINOR)], sem_rows)
            for j in range(K)
        ]
        for cp in copies:
            cp.start()
        return copies

    for b in range(NBUF):
        idx_copy(b, b).start()

    def body(g2, carry):
        for b in range(NBUF):
            g = NBUF * g2 + b
            idx_copy(g, b).wait()

            @pl.when(g2 > 0)
            def _():
                write_copy(g, b).wait()

            gathers = fire_gathers(b)
            for cp in gathers:
                cp.wait()

            @pl.when(g2 < n_chunks // NBUF - 1)
            def _():
                idx_copy(g + NBUF, b).start()

            write_copy(g, b).start()
        return carry

    lax.fori_loop(0, n_chunks // NBUF, body, 0)

    for b in range(NBUF):
        write_copy(n_chunks - NBUF + b, b).wait()


def kernel(word, pos, word_table, pos_table):
    b, l = word.shape
    bt = b * l
    per_tile = bt // NW
    vocab = word_table.shape[0]
    nblk = pl.cdiv(vocab, TBLK)
    wt_pad = pl.pallas_call(
        _transpose_pad_kernel,
        grid=(nblk,),
        in_specs=[pl.BlockSpec((WORD_DIM, TBLK), lambda i: (0, i))],
        out_specs=pl.BlockSpec((TH, IDX_MINOR), lambda i: (i, 0)),
        out_shape=jax.ShapeDtypeStruct((nblk * TH, IDX_MINOR), jnp.float32),
    )(word_table.T)
    wt_view = wt_pad.reshape(nblk * TBLK, WORD_DIM)
    wv = word.astype(jnp.int32)
    wj = wv % TBLK
    word_flat = ((wv // TBLK) * TBLK + (wj % TH) * 2 + wj // TH).reshape(
        bt // IDX_MINOR, IDX_MINOR)

    pos_size = pos_table.shape[0]
    ptab_rep = jnp.broadcast_to(
        pos_table[None], (NW,) + pos_table.shape).reshape(NW * pos_size,
                                                          pos_table.shape[1])
    rows_per_worker = per_tile // l
    rep_off = (jnp.arange(b, dtype=jnp.int32) // rows_per_worker) * pos_size
    pos_flat = (pos.astype(jnp.int32) + rep_off[:, None]).reshape(
        bt // IDX_MINOR, IDX_MINOR)

    mesh = plsc.VectorSubcoreMesh(core_axis_name="c", subcore_axis_name="s")
    scratch = [
        pltpu.VMEM((NBUF, K, IDX_MINOR), jnp.int32),
        pltpu.VMEM((NBUF, CHUNK, WORD_DIM), jnp.float32),
        pltpu.SemaphoreType.DMA((NBUF,)),
        pltpu.SemaphoreType.DMA,
        pltpu.SemaphoreType.DMA((NBUF,)),
    ]
    pos_out = pl.kernel(
        functools.partial(_half_kernel, bt, WORD_DIM),
        out_type=jax.ShapeDtypeStruct((bt, OUT_DIM), jnp.float32),
        mesh=mesh,
        compiler_params=pltpu.CompilerParams(use_tc_tiling_on_sc=False),
        scratch_types=scratch,
        name="pos_half",
    )(pos_flat, ptab_rep)
    out_ref = jax.new_ref(pos_out)
    pl.kernel(
        functools.partial(_half_kernel, bt, 0),
        out_type=(),
        mesh=mesh,
        compiler_params=pltpu.CompilerParams(use_tc_tiling_on_sc=False),
        scratch_types=scratch,
        name="word_half",
    )(word_flat, wt_view, out_ref)
    return out_ref[...].reshape(b, l, OUT_DIM)
